# baseline (speedup 1.0000x reference)
import jax
import jax.numpy as jnp
from jax import lax
from jax.experimental import pallas as pl
from jax.experimental.pallas import tpu as pltpu
from jax.experimental.pallas import tpu_sc as plsc

SEQ_LEN = 4096
D_MODEL = 1024
BATCH = 4

_info = plsc.get_sparse_core_info()
NUM_CORES = _info.num_cores
NUM_SUBCORES = _info.num_subcores
NUM_WORKERS = NUM_CORES * NUM_SUBCORES
LANES = _info.num_lanes

ROWS_PER_WORKER = SEQ_LEN // NUM_WORKERS
CHUNK_ROWS = 4
CHUNK_WORDS = CHUNK_ROWS * D_MODEL
NUM_CHUNKS = ROWS_PER_WORKER // CHUNK_ROWS


def _body(x_hbm, t_hbm, out_hbm,
          ib0, ib1, ib2, ib3, ob0, ob1, ob2, ob3, tb0, tb1,
          in_s0, in_s1, in_s2, in_s3,
          out_s0, out_s1, out_s2, out_s3, t_s0, t_s1):
    wid = lax.axis_index("s") * NUM_CORES + lax.axis_index("c")
    base_row = wid * ROWS_PER_WORKER

    ibufs = (ib0, ib1, ib2, ib3)
    obufs = (ob0, ob1, ob2, ob3)
    tbufs = (tb0, tb1)
    in_sems = (in_s0, in_s1, in_s2, in_s3)
    out_sems = (out_s0, out_s1, out_s2, out_s3)
    t_sems = (t_s0, t_s1)

    def t_slice(chunk):
        return t_hbm.at[pl.ds(base_row + chunk * CHUNK_ROWS, CHUNK_ROWS), :]

    def x_slice(chunk, b):
        return x_hbm.at[b, pl.ds(base_row + chunk * CHUNK_ROWS, CHUNK_ROWS), :]

    def o_slice(chunk, b):
        return out_hbm.at[b, pl.ds(base_row + chunk * CHUNK_ROWS, CHUNK_ROWS), :]

    pltpu.make_async_copy(t_slice(0), tb0, t_s0).start()
    for b in range(BATCH):
        pltpu.make_async_copy(x_slice(0, b), ibufs[b], in_sems[b]).start()
    pltpu.make_async_copy(t_slice(1), tb1, t_s1).start()

    def chunk_pair(it, _):
        for cp in (0, 1):
            chunk = 2 * it + cp
            pltpu.make_async_copy(t_slice(chunk), tbufs[cp], t_sems[cp]).wait()

            for h in (0, 1):
                b0, b1 = 2 * h, 2 * h + 1
                pltpu.make_async_copy(x_slice(chunk, b0), ibufs[b0],
                                      in_sems[b0]).wait()
                pltpu.make_async_copy(x_slice(chunk, b1), ibufs[b1],
                                      in_sems[b1]).wait()

                def wait_out():
                    pltpu.make_async_copy(obufs[b0], o_slice(chunk - 1, b0),
                                          out_sems[b0]).wait()
                    pltpu.make_async_copy(obufs[b1], o_slice(chunk - 1, b1),
                                          out_sems[b1]).wait()

                if cp == 0:
                    pl.when(it > 0)(wait_out)
                else:
                    wait_out()

                ia, ic = ibufs[b0], ibufs[b1]
                oa, oc = obufs[b0], obufs[b1]
                tb = tbufs[cp]

                @plsc.parallel_loop(0, CHUNK_WORDS, LANES, unroll=16)
                def add_body(i):
                    r = i // D_MODEL
                    c = i % D_MODEL
                    sl = pl.ds(c, LANES)
                    tv = tb[r, sl]
                    oa[r, sl] = ia[r, sl] + tv
                    oc[r, sl] = ic[r, sl] + tv

                pltpu.make_async_copy(obufs[b0], o_slice(chunk, b0),
                                      out_sems[b0]).start()
                pltpu.make_async_copy(obufs[b1], o_slice(chunk, b1),
                                      out_sems[b1]).start()

                def start_in():
                    pltpu.make_async_copy(x_slice(chunk + 1, b0), ibufs[b0],
                                          in_sems[b0]).start()
                    pltpu.make_async_copy(x_slice(chunk + 1, b1), ibufs[b1],
                                          in_sems[b1]).start()

                if cp == 0:
                    start_in()
                else:
                    pl.when(chunk < NUM_CHUNKS - 1)(start_in)

            def start_t():
                pltpu.make_async_copy(t_slice(chunk + 2), tbufs[cp],
                                      t_sems[cp]).start()

            pl.when(chunk < NUM_CHUNKS - 2)(start_t)
        return ()

    lax.fori_loop(0, NUM_CHUNKS // 2, chunk_pair, ())

    for b in range(BATCH):
        pltpu.make_async_copy(obufs[b], o_slice(NUM_CHUNKS - 1, b),
                              out_sems[b]).wait()


@jax.jit
def _pos_emb_add(x, t):
    mesh = plsc.VectorSubcoreMesh(core_axis_name="c", subcore_axis_name="s")
    buf = pltpu.VMEM((CHUNK_ROWS, D_MODEL), jnp.float32)
    sem = pltpu.SemaphoreType.DMA
    return pl.kernel(
        _body,
        out_type=jax.ShapeDtypeStruct((BATCH, SEQ_LEN, D_MODEL), jnp.float32),
        mesh=mesh,
        scratch_types=[buf] * 10 + [sem] * 10,
    )(x, t)


def kernel(inputs, pos_table):
    return _pos_emb_add(inputs, pos_table)


# speedup vs baseline: 1.0915x; 1.0915x over previous
import jax
import jax.numpy as jnp
from jax import lax
from jax.experimental import pallas as pl
from jax.experimental.pallas import tpu as pltpu
from jax.experimental.pallas import tpu_sc as plsc

SEQ_LEN = 4096
D_MODEL = 1024
BATCH = 4

_info = plsc.get_sparse_core_info()
NUM_CORES = _info.num_cores
NUM_SUBCORES = _info.num_subcores
NUM_WORKERS = NUM_CORES * NUM_SUBCORES
LANES = _info.num_lanes

ROWS_PER_WORKER = SEQ_LEN // NUM_WORKERS
CHUNK_ROWS = 8
CHUNK_WORDS = CHUNK_ROWS * D_MODEL
NUM_CHUNKS = ROWS_PER_WORKER // CHUNK_ROWS


def _body(x_hbm, t_hbm, out_hbm,
          ib0, ib1, ib2, ib3, ob0, ob1, ob2, ob3, tb0, tb1,
          in_s0, in_s1, in_s2, in_s3,
          out_s0, out_s1, out_s2, out_s3, t_s0, t_s1):
    wid = lax.axis_index("s") * NUM_CORES + lax.axis_index("c")
    base_row = wid * ROWS_PER_WORKER

    ibufs = (ib0, ib1, ib2, ib3)
    obufs = (ob0, ob1, ob2, ob3)
    tbufs = (tb0, tb1)
    in_sems = (in_s0, in_s1, in_s2, in_s3)
    out_sems = (out_s0, out_s1, out_s2, out_s3)
    t_sems = (t_s0, t_s1)

    def t_slice(chunk):
        return t_hbm.at[pl.ds(base_row + chunk * CHUNK_ROWS, CHUNK_ROWS), :]

    def x_slice(chunk, b):
        return x_hbm.at[b, pl.ds(base_row + chunk * CHUNK_ROWS, CHUNK_ROWS), :]

    def o_slice(chunk, b):
        return out_hbm.at[b, pl.ds(base_row + chunk * CHUNK_ROWS, CHUNK_ROWS), :]

    pltpu.make_async_copy(t_slice(0), tb0, t_s0).start()
    for b in range(BATCH):
        pltpu.make_async_copy(x_slice(0, b), ibufs[b], in_sems[b]).start()
    pltpu.make_async_copy(t_slice(1), tb1, t_s1).start()

    def chunk_pair(it, _):
        for cp in (0, 1):
            chunk = 2 * it + cp
            pltpu.make_async_copy(t_slice(chunk), tbufs[cp], t_sems[cp]).wait()

            for h in (0, 1):
                b0, b1 = 2 * h, 2 * h + 1
                pltpu.make_async_copy(x_slice(chunk, b0), ibufs[b0],
                                      in_sems[b0]).wait()
                pltpu.make_async_copy(x_slice(chunk, b1), ibufs[b1],
                                      in_sems[b1]).wait()

                def wait_out():
                    pltpu.make_async_copy(obufs[b0], o_slice(chunk - 1, b0),
                                          out_sems[b0]).wait()
                    pltpu.make_async_copy(obufs[b1], o_slice(chunk - 1, b1),
                                          out_sems[b1]).wait()

                if cp == 0:
                    pl.when(it > 0)(wait_out)
                else:
                    wait_out()

                ia, ic = ibufs[b0], ibufs[b1]
                oa, oc = obufs[b0], obufs[b1]
                tb = tbufs[cp]

                @plsc.parallel_loop(0, CHUNK_WORDS, LANES, unroll=16)
                def add_body(i):
                    r = i // D_MODEL
                    c = i % D_MODEL
                    sl = pl.ds(c, LANES)
                    tv = tb[r, sl]
                    oa[r, sl] = ia[r, sl] + tv
                    oc[r, sl] = ic[r, sl] + tv

                pltpu.make_async_copy(obufs[b0], o_slice(chunk, b0),
                                      out_sems[b0]).start()
                pltpu.make_async_copy(obufs[b1], o_slice(chunk, b1),
                                      out_sems[b1]).start()

                def start_in():
                    pltpu.make_async_copy(x_slice(chunk + 1, b0), ibufs[b0],
                                          in_sems[b0]).start()
                    pltpu.make_async_copy(x_slice(chunk + 1, b1), ibufs[b1],
                                          in_sems[b1]).start()

                if cp == 0:
                    start_in()
                else:
                    pl.when(chunk < NUM_CHUNKS - 1)(start_in)

            def start_t():
                pltpu.make_async_copy(t_slice(chunk + 2), tbufs[cp],
                                      t_sems[cp]).start()

            pl.when(chunk < NUM_CHUNKS - 2)(start_t)
        return ()

    lax.fori_loop(0, NUM_CHUNKS // 2, chunk_pair, ())

    for b in range(BATCH):
        pltpu.make_async_copy(obufs[b], o_slice(NUM_CHUNKS - 1, b),
                              out_sems[b]).wait()


@jax.jit
def _pos_emb_add(x, t):
    mesh = plsc.VectorSubcoreMesh(core_axis_name="c", subcore_axis_name="s")
    buf = pltpu.VMEM((CHUNK_ROWS, D_MODEL), jnp.float32)
    sem = pltpu.SemaphoreType.DMA
    return pl.kernel(
        _body,
        out_type=jax.ShapeDtypeStruct((BATCH, SEQ_LEN, D_MODEL), jnp.float32),
        mesh=mesh,
        scratch_types=[buf] * 10 + [sem] * 10,
    )(x, t)


def kernel(inputs, pos_table):
    return _pos_emb_add(inputs, pos_table)
